# baseline (speedup 1.0000x reference)
import functools

import jax
import jax.numpy as jnp
from jax import lax
from jax.experimental import pallas as pl
from jax.experimental.pallas import tpu as pltpu
from jax.experimental.pallas import tpu_sc as plsc

_C = 64
_D = 128
_N = 320000
_R = 32000

_NCORE = 2
_NSUB = 16
_NW = _NCORE * _NSUB
_RPW = _N // _NW
_CH = 80
_NCH = _RPW // _CH
_CW = 128


def _sc_seg_kernel(x_hbm, lab_hbm, sums_out, counts_out,
                   lab_v, x_v, ones_v, stage_v, cstage_v, sums_sh, counts_sh):
    c = lax.axis_index("c")
    s = lax.axis_index("s")
    wid = s * _NCORE + c
    base = wid * _RPW

    zeros16 = jnp.zeros((16,), jnp.float32)
    ones16 = jnp.ones((16,), jnp.float32)

    def _fill(ref, rows, val16):
        def _body(k, _):
            r = k // (_D // 16)
            col = (k % (_D // 16)) * 16
            ref[r, pl.ds(col, 16)] = val16
            return 0
        lax.fori_loop(0, rows * (_D // 16), _body, 0)

    _fill(ones_v, _CH, ones16)
    _fill(stage_v, _C, zeros16)
    _fill(cstage_v, _C, zeros16)

    pltpu.sync_copy(stage_v, sums_sh.at[s])
    pltpu.sync_copy(cstage_v, counts_sh.at[s])

    pltpu.sync_copy(lab_hbm.at[wid], lab_v)

    def _chunk(j, _):
        start = pl.multiple_of(base + j * _CH, _CH)
        pltpu.sync_copy(x_hbm.at[pl.ds(start, _CH)], x_v)
        pltpu.sync_copy(x_v, sums_sh.at[s].at[lab_v.at[j]], add=True)
        pltpu.sync_copy(ones_v, counts_sh.at[s].at[lab_v.at[j]], add=True)
        return 0
    lax.fori_loop(0, _NCH, _chunk, 0)

    pltpu.sync_copy(sums_sh.at[s], sums_out.at[c, s])
    pltpu.sync_copy(counts_sh.at[s], counts_out.at[c, s])


def _sc_segment_sums(x, labels):
    lab2 = labels.reshape(_NW, _NCH, _CH)
    mesh = plsc.VectorSubcoreMesh(core_axis_name="c", subcore_axis_name="s")
    run = functools.partial(
        pl.kernel,
        mesh=mesh,
        out_type=[
            jax.ShapeDtypeStruct((_NCORE, _NSUB, _C, _D), jnp.float32),
            jax.ShapeDtypeStruct((_NCORE, _NSUB, _C, _CW), jnp.float32),
        ],
        scratch_types=[
            pltpu.VMEM((_NCH, _CH), jnp.int32),
            pltpu.VMEM((_CH, _D), jnp.float32),
            pltpu.VMEM((_CH, _CW), jnp.float32),
            pltpu.VMEM((_C, _D), jnp.float32),
            pltpu.VMEM((_C, _CW), jnp.float32),
            pltpu.MemorySpace.VMEM_SHARED((_NSUB, _C, _D), jnp.float32),
            pltpu.MemorySpace.VMEM_SHARED((_NSUB, _C, _CW), jnp.float32),
        ],
    )(_sc_seg_kernel)
    return run(x, lab2)


def _dist_kernel(sums_ref, counts_ref, x_ref, lab_ref, out_ref,
                 cent_ref, counts_v, sums_v, acc_ref, nc_ref):
    i = pl.program_id(0)
    nb = pl.num_programs(0)

    @pl.when(i == 0)
    def _():
        counts = jnp.sum(counts_ref[...], axis=0)[:, 0:1]
        sums = jnp.sum(sums_ref[...], axis=0)
        counts_v[...] = counts
        sums_v[...] = sums
        nc_ref[0] = jnp.sum((counts > 0.0).astype(jnp.int32))
        cent_ref[...] = sums / jnp.maximum(counts, 1.0)
        acc_ref[0] = 0.0

    nc = nc_ref[0]
    lab = lab_ref[0, 0, :]
    g = jnp.minimum(lab, nc - 1)
    onehot = (g[:, None] ==
              lax.broadcasted_iota(jnp.int32, (_R, _C), 1)
              ).astype(jnp.float32)
    cent_rows = jnp.dot(onehot, cent_ref[...],
                        preferred_element_type=jnp.float32)
    diff = x_ref[...] - cent_rows
    ones_mat = jnp.ones((_D, _D), dtype=jnp.float32)
    e_dup = jnp.dot(diff * diff, ones_mat,
                    preferred_element_type=jnp.float32)
    dist = e_dup * lax.rsqrt(e_dup + 1e-37)
    acc_ref[0] += jnp.sum(dist) * (1.0 / float(_D))

    @pl.when(i == nb - 1)
    def _():
        nc_f = nc.astype(jnp.float32)
        counts = counts_v[...]
        gm = jnp.sum(sums_v[...], axis=0, keepdims=True) / float(_N)
        dc = cent_ref[...] - gm
        e_c = jnp.sum(dc * dc, axis=1, keepdims=True)
        d = e_c * lax.rsqrt(e_c + 1e-37)
        cidx = lax.broadcasted_iota(jnp.int32, (_C, 1), 0)
        between = (jnp.sum(jnp.where(cidx < nc, counts * d, 0.0))
                   / (nc_f - 1.0))
        within = acc_ref[0] / (float(_N) - nc_f)
        out_ref[...] = jnp.full((1, 1), between / within, dtype=jnp.float32)


def _distance_pass(sums2, counts2, x, labels):
    nb = _N // _R
    lab3 = labels.reshape(nb, 1, _R)
    sums3 = sums2.reshape(_NW, _C, _D)
    counts3 = counts2.reshape(_NW, _C, _CW)
    return pl.pallas_call(
        _dist_kernel,
        grid=(nb,),
        in_specs=[
            pl.BlockSpec((_NW, _C, _D), lambda i: (0, 0, 0)),
            pl.BlockSpec((_NW, _C, _CW), lambda i: (0, 0, 0)),
            pl.BlockSpec((_R, _D), lambda i: (i, 0)),
            pl.BlockSpec((1, 1, _R), lambda i: (i, 0, 0)),
        ],
        out_specs=pl.BlockSpec((1, 1), lambda i: (0, 0)),
        out_shape=jax.ShapeDtypeStruct((1, 1), jnp.float32),
        scratch_shapes=[
            pltpu.VMEM((_C, _D), jnp.float32),
            pltpu.VMEM((_C, 1), jnp.float32),
            pltpu.VMEM((_C, _D), jnp.float32),
            pltpu.SMEM((1,), jnp.float32),
            pltpu.SMEM((1,), jnp.int32),
        ],
    )(sums3, counts3, x, lab3)


def kernel(Attributes, cluster_labels):
    labels = cluster_labels[0]
    sums2, counts2 = _sc_segment_sums(Attributes, labels)
    loss = _distance_pass(sums2, counts2, Attributes, labels)
    return loss.reshape(1)


# speedup vs baseline: 1.6849x; 1.6849x over previous
import functools

import jax
import jax.numpy as jnp
from jax import lax
from jax.experimental import pallas as pl
from jax.experimental.pallas import tpu as pltpu
from jax.experimental.pallas import tpu_sc as plsc

_C = 64
_D = 128
_N = 320000
_R = 32000

_NCORE = 2
_NSUB = 16
_NW = _NCORE * _NSUB
_RPW = _N // _NW
_CH = 40
_NCH = _RPW // _CH
_G = 5
_NRING = 2 * _G
_NRD = _NCH // _G


def _sc_seg_kernel(x_hbm, lab_hbm, sums_out,
                   lab_v, stage_v, xring_v, sums_sh, gsem, ssem):
    c = lax.axis_index("c")
    s = lax.axis_index("s")
    wid = s * _NCORE + c
    base = wid * _RPW

    zeros16 = jnp.zeros((16,), jnp.float32)

    def _fill_zero(k, _):
        r = k // (_D // 16)
        col = (k % (_D // 16)) * 16
        stage_v[r, pl.ds(col, 16)] = zeros16
        return 0
    lax.fori_loop(0, _C * (_D // 16), _fill_zero, 0)

    pltpu.sync_copy(stage_v, sums_sh.at[s])

    pltpu.sync_copy(lab_hbm.at[wid], lab_v)

    def _gather(j, slot):
        start = pl.multiple_of(base + j * _CH, _CH)
        return pltpu.make_async_copy(
            x_hbm.at[pl.ds(start, _CH)], xring_v.at[slot], gsem.at[slot])

    def _scatter(j, slot):
        return pltpu.make_async_copy(
            xring_v.at[slot], sums_sh.at[s].at[lab_v.at[j]], ssem.at[slot])

    for b in range(_G):
        _gather(b, b).start()

    def _round(r, _):
        g = lax.rem(r, 2)
        other = (1 - g) * _G

        @pl.when(r + 1 < _NRD)
        def _():
            for b in range(_G):
                slot = other + b
                j2 = (r + 1) * _G + b

                @pl.when(r >= 1)
                def _():
                    _scatter(j2 - _NRING, slot).wait()
                _gather(j2, slot).start()

        for b in range(_G):
            slot = g * _G + b
            j = r * _G + b
            _gather(j, slot).wait()
            _scatter(j, slot).start(add=True)
        return 0
    lax.fori_loop(0, _NRD, _round, 0)

    for slot in range(_NRING):
        _scatter(0, slot).wait()

    pltpu.sync_copy(sums_sh.at[s], sums_out.at[c, s])


def _sc_segment_sums(x, labels):
    lab2 = labels.reshape(_NW, _NCH, _CH)
    mesh = plsc.VectorSubcoreMesh(core_axis_name="c", subcore_axis_name="s")
    run = functools.partial(
        pl.kernel,
        mesh=mesh,
        out_type=jax.ShapeDtypeStruct((_NCORE, _NSUB, _C, _D), jnp.float32),
        scratch_types=[
            pltpu.VMEM((_NCH, _CH), jnp.int32),
            pltpu.VMEM((_C, _D), jnp.float32),
            pltpu.VMEM((_NRING, _CH, _D), jnp.float32),
            pltpu.MemorySpace.VMEM_SHARED((_NSUB, _C, _D), jnp.float32),
            pltpu.SemaphoreType.DMA((_NRING,)),
            pltpu.SemaphoreType.DMA((_NRING,)),
        ],
    )(_sc_seg_kernel)
    return run(x, lab2)


def _dist_kernel(sums_ref, x_ref, lab_ref, out_ref,
                 cent_ref, counts_v, sums_v, acc_ref, nc_ref):
    p = pl.program_id(0)
    i = pl.program_id(1)
    nb = pl.num_programs(1)
    lab = lab_ref[0, 0, :]

    @pl.when((p == 0) & (i == 0))
    def _():
        counts_v[...] = jnp.zeros_like(counts_v)

    @pl.when(p == 0)
    def _():
        clusters = lax.broadcasted_iota(jnp.int32, (_C, _R), 0)
        onehot_t = (clusters == lab[None, :]).astype(jnp.float32)
        counts_v[...] += jnp.sum(onehot_t, axis=1, keepdims=True)

    @pl.when((p == 1) & (i == 0))
    def _():
        counts = counts_v[...]
        sums = jnp.sum(sums_ref[...], axis=0)
        sums_v[...] = sums
        nc_ref[0] = jnp.sum((counts > 0.0).astype(jnp.int32))
        cent_ref[...] = sums / jnp.maximum(counts, 1.0)
        acc_ref[0] = 0.0

    @pl.when(p == 1)
    def _():
        nc = nc_ref[0]
        g = jnp.minimum(lab, nc - 1)
        onehot = (g[:, None] ==
                  lax.broadcasted_iota(jnp.int32, (_R, _C), 1)
                  ).astype(jnp.float32)
        cent_rows = jnp.dot(onehot, cent_ref[...],
                            preferred_element_type=jnp.float32)
        diff = x_ref[...] - cent_rows
        ones_mat = jnp.ones((_D, _D), dtype=jnp.float32)
        e_dup = jnp.dot(diff * diff, ones_mat,
                        preferred_element_type=jnp.float32)
        dist = e_dup * lax.rsqrt(e_dup + 1e-37)
        acc_ref[0] += jnp.sum(dist) * (1.0 / float(_D))

        @pl.when(i == nb - 1)
        def _():
            nc_f = nc.astype(jnp.float32)
            counts = counts_v[...]
            gm = jnp.sum(sums_v[...], axis=0, keepdims=True) / float(_N)
            dc = cent_ref[...] - gm
            e_c = jnp.sum(dc * dc, axis=1, keepdims=True)
            d = e_c * lax.rsqrt(e_c + 1e-37)
            cidx = lax.broadcasted_iota(jnp.int32, (_C, 1), 0)
            between = (jnp.sum(jnp.where(cidx < nc, counts * d, 0.0))
                       / (nc_f - 1.0))
            within = acc_ref[0] / (float(_N) - nc_f)
            out_ref[...] = jnp.full((1, 1), between / within,
                                    dtype=jnp.float32)


def _distance_pass(sums2, x, labels):
    nb = _N // _R
    lab3 = labels.reshape(nb, 1, _R)
    sums3 = sums2.reshape(_NW, _C, _D)
    return pl.pallas_call(
        _dist_kernel,
        grid=(2, nb),
        in_specs=[
            pl.BlockSpec((_NW, _C, _D), lambda p, i: (0, 0, 0)),
            pl.BlockSpec((_R, _D), lambda p, i: (p * i, 0)),
            pl.BlockSpec((1, 1, _R), lambda p, i: (i, 0, 0)),
        ],
        out_specs=pl.BlockSpec((1, 1), lambda p, i: (0, 0)),
        out_shape=jax.ShapeDtypeStruct((1, 1), jnp.float32),
        scratch_shapes=[
            pltpu.VMEM((_C, _D), jnp.float32),
            pltpu.VMEM((_C, 1), jnp.float32),
            pltpu.VMEM((_C, _D), jnp.float32),
            pltpu.SMEM((1,), jnp.float32),
            pltpu.SMEM((1,), jnp.int32),
        ],
    )(sums3, x, lab3)


def kernel(Attributes, cluster_labels):
    labels = cluster_labels[0]
    sums2 = _sc_segment_sums(Attributes, labels)
    loss = _distance_pass(sums2, Attributes, labels)
    return loss.reshape(1)


# speedup vs baseline: 1.8196x; 1.0799x over previous
import functools

import jax
import jax.numpy as jnp
from jax import lax
from jax.experimental import pallas as pl
from jax.experimental.pallas import tpu as pltpu
from jax.experimental.pallas import tpu_sc as plsc

_C = 64
_D = 128
_N = 320000
_R = 32000

_NCORE = 2
_NSUB = 16
_NW = _NCORE * _NSUB
_NTC = 64000
_NTCB = _NTC // _R
_RPW = (_N - _NTC) // _NW
_CH = 40
_NCH = _RPW // _CH
_G = 5
_NRING = 2 * _G
_NRD = _NCH // _G


def _sc_seg_kernel(x_hbm, lab_hbm, sums_out,
                   lab_v, stage_v, xring_v, sums_sh, gsem, ssem):
    c = lax.axis_index("c")
    s = lax.axis_index("s")
    wid = s * _NCORE + c
    base = _NTC + wid * _RPW

    zeros16 = jnp.zeros((16,), jnp.float32)

    def _fill_zero(k, _):
        r = k // (_D // 16)
        col = (k % (_D // 16)) * 16
        stage_v[r, pl.ds(col, 16)] = zeros16
        return 0
    lax.fori_loop(0, _C * (_D // 16), _fill_zero, 0)

    pltpu.sync_copy(stage_v, sums_sh.at[s])

    pltpu.sync_copy(lab_hbm.at[wid], lab_v)

    def _gather(j, slot):
        start = pl.multiple_of(base + j * _CH, _CH)
        return pltpu.make_async_copy(
            x_hbm.at[pl.ds(start, _CH)], xring_v.at[slot], gsem.at[slot])

    def _scatter(j, slot):
        return pltpu.make_async_copy(
            xring_v.at[slot], sums_sh.at[s].at[lab_v.at[j]], ssem.at[slot])

    for b in range(_G):
        _gather(b, b).start()

    def _round(r, _):
        g = lax.rem(r, 2)
        other = (1 - g) * _G

        @pl.when(r + 1 < _NRD)
        def _():
            for b in range(_G):
                slot = other + b
                j2 = (r + 1) * _G + b

                @pl.when(r >= 1)
                def _():
                    _scatter(j2 - _NRING, slot).wait()
                _gather(j2, slot).start()

        for b in range(_G):
            slot = g * _G + b
            j = r * _G + b
            _gather(j, slot).wait()
            _scatter(j, slot).start(add=True)
        return 0
    lax.fori_loop(0, _NRD, _round, 0)

    for slot in range(_NRING):
        _scatter(0, slot).wait()

    pltpu.sync_copy(sums_sh.at[s], sums_out.at[c, s])


def _sc_segment_sums(x, labels):
    lab2 = labels[_NTC:].reshape(_NW, _NCH, _CH)
    mesh = plsc.VectorSubcoreMesh(core_axis_name="c", subcore_axis_name="s")
    run = functools.partial(
        pl.kernel,
        mesh=mesh,
        out_type=jax.ShapeDtypeStruct((_NCORE, _NSUB, _C, _D), jnp.float32),
        scratch_types=[
            pltpu.VMEM((_NCH, _CH), jnp.int32),
            pltpu.VMEM((_C, _D), jnp.float32),
            pltpu.VMEM((_NRING, _CH, _D), jnp.float32),
            pltpu.MemorySpace.VMEM_SHARED((_NSUB, _C, _D), jnp.float32),
            pltpu.SemaphoreType.DMA((_NRING,)),
            pltpu.SemaphoreType.DMA((_NRING,)),
        ],
    )(_sc_seg_kernel)
    return run(x, lab2)


def _tc_sega_kernel(x_ref, lab_ref, sums_ref, counts_ref):
    i = pl.program_id(0)
    lab = lab_ref[0, 0, :]

    @pl.when(i == 0)
    def _():
        sums_ref[...] = jnp.zeros_like(sums_ref)
        counts_ref[...] = jnp.zeros_like(counts_ref)

    clusters = lax.broadcasted_iota(jnp.int32, (_C, _R), 0)
    onehot_t = (clusters == lab[None, :]).astype(jnp.float32)
    counts_ref[...] += jnp.sum(onehot_t, axis=1, keepdims=True)

    @pl.when(i < _NTCB)
    def _():
        sums_ref[...] += jnp.dot(onehot_t, x_ref[...],
                                 preferred_element_type=jnp.float32)


def _tc_sega(x, labels):
    nb = _N // _R
    lab3 = labels.reshape(nb, 1, _R)
    return pl.pallas_call(
        _tc_sega_kernel,
        grid=(nb,),
        in_specs=[
            pl.BlockSpec((_R, _D), lambda i: (jnp.minimum(i, _NTCB - 1), 0)),
            pl.BlockSpec((1, 1, _R), lambda i: (i, 0, 0)),
        ],
        out_specs=[
            pl.BlockSpec((_C, _D), lambda i: (0, 0)),
            pl.BlockSpec((_C, 1), lambda i: (0, 0)),
        ],
        out_shape=[
            jax.ShapeDtypeStruct((_C, _D), jnp.float32),
            jax.ShapeDtypeStruct((_C, 1), jnp.float32),
        ],
    )(x, lab3)


def _dist_kernel(sums_ref, tcsums_ref, counts_ref, x_ref, lab_ref, out_ref,
                 cent_ref, sums_v, acc_ref, nc_ref):
    i = pl.program_id(0)
    nb = pl.num_programs(0)
    lab = lab_ref[0, 0, :]

    @pl.when(i == 0)
    def _():
        counts = counts_ref[...]
        sums = jnp.sum(sums_ref[...], axis=0) + tcsums_ref[...]
        sums_v[...] = sums
        nc_ref[0] = jnp.sum((counts > 0.0).astype(jnp.int32))
        cent_ref[...] = sums / jnp.maximum(counts, 1.0)
        acc_ref[0] = 0.0

    nc = nc_ref[0]
    g = jnp.minimum(lab, nc - 1)
    onehot = (g[:, None] ==
              lax.broadcasted_iota(jnp.int32, (_R, _C), 1)
              ).astype(jnp.float32)
    cent_rows = jnp.dot(onehot, cent_ref[...],
                        preferred_element_type=jnp.float32)
    diff = x_ref[...] - cent_rows
    ones_mat = jnp.ones((_D, _D), dtype=jnp.float32)
    e_dup = jnp.dot(diff * diff, ones_mat,
                    preferred_element_type=jnp.float32)
    dist = e_dup * lax.rsqrt(e_dup + 1e-37)
    acc_ref[0] += jnp.sum(dist) * (1.0 / float(_D))

    @pl.when(i == nb - 1)
    def _():
        nc_f = nc.astype(jnp.float32)
        counts = counts_ref[...]
        gm = jnp.sum(sums_v[...], axis=0, keepdims=True) / float(_N)
        dc = cent_ref[...] - gm
        e_c = jnp.sum(dc * dc, axis=1, keepdims=True)
        d = e_c * lax.rsqrt(e_c + 1e-37)
        cidx = lax.broadcasted_iota(jnp.int32, (_C, 1), 0)
        between = (jnp.sum(jnp.where(cidx < nc, counts * d, 0.0))
                   / (nc_f - 1.0))
        within = acc_ref[0] / (float(_N) - nc_f)
        out_ref[...] = jnp.full((1, 1), between / within, dtype=jnp.float32)


def _distance_pass(sums2, tcsums, counts, x, labels):
    nb = _N // _R
    lab3 = labels.reshape(nb, 1, _R)
    sums3 = sums2.reshape(_NW, _C, _D)
    return pl.pallas_call(
        _dist_kernel,
        grid=(nb,),
        in_specs=[
            pl.BlockSpec((_NW, _C, _D), lambda i: (0, 0, 0)),
            pl.BlockSpec((_C, _D), lambda i: (0, 0)),
            pl.BlockSpec((_C, 1), lambda i: (0, 0)),
            pl.BlockSpec((_R, _D), lambda i: (i, 0)),
            pl.BlockSpec((1, 1, _R), lambda i: (i, 0, 0)),
        ],
        out_specs=pl.BlockSpec((1, 1), lambda i: (0, 0)),
        out_shape=jax.ShapeDtypeStruct((1, 1), jnp.float32),
        scratch_shapes=[
            pltpu.VMEM((_C, _D), jnp.float32),
            pltpu.VMEM((_C, _D), jnp.float32),
            pltpu.SMEM((1,), jnp.float32),
            pltpu.SMEM((1,), jnp.int32),
        ],
    )(sums3, tcsums, counts, x, lab3)


def kernel(Attributes, cluster_labels):
    labels = cluster_labels[0]
    sums2 = _sc_segment_sums(Attributes, labels)
    tcsums, counts = _tc_sega(Attributes, labels)
    loss = _distance_pass(sums2, tcsums, counts, Attributes, labels)
    return loss.reshape(1)


# speedup vs baseline: 1.9424x; 1.0675x over previous
import functools

import jax
import jax.numpy as jnp
from jax import lax
from jax.experimental import pallas as pl
from jax.experimental.pallas import tpu as pltpu
from jax.experimental.pallas import tpu_sc as plsc

_C = 64
_D = 128
_N = 320000
_R = 32000

_NCORE = 2
_NSUB = 16
_NW = _NCORE * _NSUB
_NTC = 128000
_NTCB = _NTC // _R
_RPW = (_N - _NTC) // _NW
_CH = 40
_NCH = _RPW // _CH
_G = 5
_NRING = 2 * _G
_NRD = _NCH // _G


def _sc_seg_kernel(x_hbm, lab_hbm, sums_out,
                   lab_v, stage_v, xring_v, sums_sh, gsem, ssem):
    c = lax.axis_index("c")
    s = lax.axis_index("s")
    wid = s * _NCORE + c
    base = _NTC + wid * _RPW

    zeros16 = jnp.zeros((16,), jnp.float32)

    def _fill_zero(k, _):
        r = k // (_D // 16)
        col = (k % (_D // 16)) * 16
        stage_v[r, pl.ds(col, 16)] = zeros16
        return 0
    lax.fori_loop(0, _C * (_D // 16), _fill_zero, 0)

    pltpu.sync_copy(stage_v, sums_sh.at[s])

    pltpu.sync_copy(lab_hbm.at[wid], lab_v)

    def _gather(j, slot):
        start = pl.multiple_of(base + j * _CH, _CH)
        return pltpu.make_async_copy(
            x_hbm.at[pl.ds(start, _CH)], xring_v.at[slot], gsem.at[slot])

    def _scatter(j, slot):
        return pltpu.make_async_copy(
            xring_v.at[slot], sums_sh.at[s].at[lab_v.at[j]], ssem.at[slot])

    for b in range(_G):
        _gather(b, b).start()

    def _round(r, _):
        g = lax.rem(r, 2)
        other = (1 - g) * _G

        @pl.when(r + 1 < _NRD)
        def _():
            for b in range(_G):
                slot = other + b
                j2 = (r + 1) * _G + b

                @pl.when(r >= 1)
                def _():
                    _scatter(j2 - _NRING, slot).wait()
                _gather(j2, slot).start()

        for b in range(_G):
            slot = g * _G + b
            j = r * _G + b
            _gather(j, slot).wait()
            _scatter(j, slot).start(add=True)
        return 0
    lax.fori_loop(0, _NRD, _round, 0)

    for slot in range(_NRING):
        _scatter(0, slot).wait()

    pltpu.sync_copy(sums_sh.at[s], sums_out.at[c, s])


def _sc_segment_sums(x, labels):
    lab2 = labels[_NTC:].reshape(_NW, _NCH, _CH)
    mesh = plsc.VectorSubcoreMesh(core_axis_name="c", subcore_axis_name="s")
    run = functools.partial(
        pl.kernel,
        mesh=mesh,
        out_type=jax.ShapeDtypeStruct((_NCORE, _NSUB, _C, _D), jnp.float32),
        scratch_types=[
            pltpu.VMEM((_NCH, _CH), jnp.int32),
            pltpu.VMEM((_C, _D), jnp.float32),
            pltpu.VMEM((_NRING, _CH, _D), jnp.float32),
            pltpu.MemorySpace.VMEM_SHARED((_NSUB, _C, _D), jnp.float32),
            pltpu.SemaphoreType.DMA((_NRING,)),
            pltpu.SemaphoreType.DMA((_NRING,)),
        ],
    )(_sc_seg_kernel)
    return run(x, lab2)


def _tc_sega_kernel(x_ref, lab_ref, sums_ref, counts_ref):
    i = pl.program_id(0)
    lab = lab_ref[0, 0, :]

    @pl.when(i == 0)
    def _():
        sums_ref[...] = jnp.zeros_like(sums_ref)
        counts_ref[...] = jnp.zeros_like(counts_ref)

    clusters = lax.broadcasted_iota(jnp.int32, (_C, _R), 0)
    onehot_t = (clusters == lab[None, :]).astype(jnp.float32)
    counts_ref[...] += jnp.sum(onehot_t, axis=1, keepdims=True)

    @pl.when(i < _NTCB)
    def _():
        sums_ref[...] += jnp.dot(onehot_t, x_ref[...],
                                 preferred_element_type=jnp.float32)


def _tc_sega(x, labels):
    nb = _N // _R
    lab3 = labels.reshape(nb, 1, _R)
    return pl.pallas_call(
        _tc_sega_kernel,
        grid=(nb,),
        in_specs=[
            pl.BlockSpec((_R, _D), lambda i: (jnp.minimum(i, _NTCB - 1), 0)),
            pl.BlockSpec((1, 1, _R), lambda i: (i, 0, 0)),
        ],
        out_specs=[
            pl.BlockSpec((_C, _D), lambda i: (0, 0)),
            pl.BlockSpec((_C, 1), lambda i: (0, 0)),
        ],
        out_shape=[
            jax.ShapeDtypeStruct((_C, _D), jnp.float32),
            jax.ShapeDtypeStruct((_C, 1), jnp.float32),
        ],
    )(x, lab3)


def _dist_kernel(sums_ref, tcsums_ref, counts_ref, x_ref, lab_ref, out_ref,
                 cent_ref, sums_v, acc_ref, nc_ref):
    i = pl.program_id(0)
    nb = pl.num_programs(0)
    lab = lab_ref[0, 0, :]

    @pl.when(i == 0)
    def _():
        counts = counts_ref[...]
        sums = jnp.sum(sums_ref[...], axis=0) + tcsums_ref[...]
        sums_v[...] = sums
        nc_ref[0] = jnp.sum((counts > 0.0).astype(jnp.int32))
        cent_ref[...] = sums / jnp.maximum(counts, 1.0)
        acc_ref[0] = 0.0

    nc = nc_ref[0]
    g = jnp.minimum(lab, nc - 1)
    onehot = (g[:, None] ==
              lax.broadcasted_iota(jnp.int32, (_R, _C), 1)
              ).astype(jnp.float32)
    cent_rows = jnp.dot(onehot, cent_ref[...],
                        preferred_element_type=jnp.float32)
    diff = x_ref[...] - cent_rows
    ones_mat = jnp.ones((_D, _D), dtype=jnp.float32)
    e_dup = jnp.dot(diff * diff, ones_mat,
                    preferred_element_type=jnp.float32)
    dist = e_dup * lax.rsqrt(e_dup + 1e-37)
    acc_ref[0] += jnp.sum(dist) * (1.0 / float(_D))

    @pl.when(i == nb - 1)
    def _():
        nc_f = nc.astype(jnp.float32)
        counts = counts_ref[...]
        gm = jnp.sum(sums_v[...], axis=0, keepdims=True) / float(_N)
        dc = cent_ref[...] - gm
        e_c = jnp.sum(dc * dc, axis=1, keepdims=True)
        d = e_c * lax.rsqrt(e_c + 1e-37)
        cidx = lax.broadcasted_iota(jnp.int32, (_C, 1), 0)
        between = (jnp.sum(jnp.where(cidx < nc, counts * d, 0.0))
                   / (nc_f - 1.0))
        within = acc_ref[0] / (float(_N) - nc_f)
        out_ref[...] = jnp.full((1, 1), between / within, dtype=jnp.float32)


def _distance_pass(sums2, tcsums, counts, x, labels):
    nb = _N // _R
    lab3 = labels.reshape(nb, 1, _R)
    sums3 = sums2.reshape(_NW, _C, _D)
    return pl.pallas_call(
        _dist_kernel,
        grid=(nb,),
        in_specs=[
            pl.BlockSpec((_NW, _C, _D), lambda i: (0, 0, 0)),
            pl.BlockSpec((_C, _D), lambda i: (0, 0)),
            pl.BlockSpec((_C, 1), lambda i: (0, 0)),
            pl.BlockSpec((_R, _D), lambda i: (i, 0)),
            pl.BlockSpec((1, 1, _R), lambda i: (i, 0, 0)),
        ],
        out_specs=pl.BlockSpec((1, 1), lambda i: (0, 0)),
        out_shape=jax.ShapeDtypeStruct((1, 1), jnp.float32),
        scratch_shapes=[
            pltpu.VMEM((_C, _D), jnp.float32),
            pltpu.VMEM((_C, _D), jnp.float32),
            pltpu.SMEM((1,), jnp.float32),
            pltpu.SMEM((1,), jnp.int32),
        ],
    )(sums3, tcsums, counts, x, lab3)


def kernel(Attributes, cluster_labels):
    labels = cluster_labels[0]
    sums2 = _sc_segment_sums(Attributes, labels)
    tcsums, counts = _tc_sega(Attributes, labels)
    loss = _distance_pass(sums2, tcsums, counts, Attributes, labels)
    return loss.reshape(1)


# speedup vs baseline: 1.9777x; 1.0181x over previous
import functools

import jax
import jax.numpy as jnp
from jax import lax
from jax.experimental import pallas as pl
from jax.experimental.pallas import tpu as pltpu
from jax.experimental.pallas import tpu_sc as plsc

_C = 64
_D = 128
_N = 320000
_R = 32000

_NCORE = 2
_NSUB = 16
_NW = _NCORE * _NSUB
_NTC = 192000
_NTCB = _NTC // _R
_RPW = (_N - _NTC) // _NW
_CH = 40
_NCH = _RPW // _CH
_G = 5
_NRING = 2 * _G
_NRD = _NCH // _G


def _sc_seg_kernel(x_hbm, lab_hbm, sums_out,
                   lab_v, stage_v, xring_v, sums_sh, gsem, ssem):
    c = lax.axis_index("c")
    s = lax.axis_index("s")
    wid = s * _NCORE + c
    base = _NTC + wid * _RPW

    zeros16 = jnp.zeros((16,), jnp.float32)

    def _fill_zero(k, _):
        r = k // (_D // 16)
        col = (k % (_D // 16)) * 16
        stage_v[r, pl.ds(col, 16)] = zeros16
        return 0
    lax.fori_loop(0, _C * (_D // 16), _fill_zero, 0)

    pltpu.sync_copy(stage_v, sums_sh.at[s])

    pltpu.sync_copy(lab_hbm.at[wid], lab_v)

    def _gather(j, slot):
        start = pl.multiple_of(base + j * _CH, _CH)
        return pltpu.make_async_copy(
            x_hbm.at[pl.ds(start, _CH)], xring_v.at[slot], gsem.at[slot])

    def _scatter(j, slot):
        return pltpu.make_async_copy(
            xring_v.at[slot], sums_sh.at[s].at[lab_v.at[j]], ssem.at[slot])

    for b in range(_G):
        _gather(b, b).start()

    def _round(r, _):
        g = lax.rem(r, 2)
        other = (1 - g) * _G

        @pl.when(r + 1 < _NRD)
        def _():
            for b in range(_G):
                slot = other + b
                j2 = (r + 1) * _G + b

                @pl.when(r >= 1)
                def _():
                    _scatter(j2 - _NRING, slot).wait()
                _gather(j2, slot).start()

        for b in range(_G):
            slot = g * _G + b
            j = r * _G + b
            _gather(j, slot).wait()
            _scatter(j, slot).start(add=True)
        return 0
    lax.fori_loop(0, _NRD, _round, 0)

    for slot in range(_NRING):
        _scatter(0, slot).wait()

    pltpu.sync_copy(sums_sh.at[s], sums_out.at[c, s])


def _sc_segment_sums(x, labels):
    lab2 = labels[_NTC:].reshape(_NW, _NCH, _CH)
    mesh = plsc.VectorSubcoreMesh(core_axis_name="c", subcore_axis_name="s")
    run = functools.partial(
        pl.kernel,
        mesh=mesh,
        out_type=jax.ShapeDtypeStruct((_NCORE, _NSUB, _C, _D), jnp.float32),
        scratch_types=[
            pltpu.VMEM((_NCH, _CH), jnp.int32),
            pltpu.VMEM((_C, _D), jnp.float32),
            pltpu.VMEM((_NRING, _CH, _D), jnp.float32),
            pltpu.MemorySpace.VMEM_SHARED((_NSUB, _C, _D), jnp.float32),
            pltpu.SemaphoreType.DMA((_NRING,)),
            pltpu.SemaphoreType.DMA((_NRING,)),
        ],
    )(_sc_seg_kernel)
    return run(x, lab2)


def _tc_sega_kernel(x_ref, lab_ref, sums_ref, counts_ref):
    i = pl.program_id(0)
    lab = lab_ref[0, 0, :]

    @pl.when(i == 0)
    def _():
        sums_ref[...] = jnp.zeros_like(sums_ref)
        counts_ref[...] = jnp.zeros_like(counts_ref)

    clusters = lax.broadcasted_iota(jnp.int32, (_C, _R), 0)
    onehot_t = (clusters == lab[None, :]).astype(jnp.float32)
    counts_ref[...] += jnp.sum(onehot_t, axis=1, keepdims=True)

    @pl.when(i < _NTCB)
    def _():
        sums_ref[...] += jnp.dot(onehot_t, x_ref[...],
                                 preferred_element_type=jnp.float32)


def _tc_sega(x, labels):
    nb = _N // _R
    lab3 = labels.reshape(nb, 1, _R)
    return pl.pallas_call(
        _tc_sega_kernel,
        grid=(nb,),
        in_specs=[
            pl.BlockSpec((_R, _D), lambda i: (jnp.minimum(i, _NTCB - 1), 0)),
            pl.BlockSpec((1, 1, _R), lambda i: (i, 0, 0)),
        ],
        out_specs=[
            pl.BlockSpec((_C, _D), lambda i: (0, 0)),
            pl.BlockSpec((_C, 1), lambda i: (0, 0)),
        ],
        out_shape=[
            jax.ShapeDtypeStruct((_C, _D), jnp.float32),
            jax.ShapeDtypeStruct((_C, 1), jnp.float32),
        ],
    )(x, lab3)


def _dist_kernel(sums_ref, tcsums_ref, counts_ref, x_ref, lab_ref, out_ref,
                 cent_ref, sums_v, acc_ref, nc_ref):
    i = pl.program_id(0)
    nb = pl.num_programs(0)
    lab = lab_ref[0, 0, :]

    @pl.when(i == 0)
    def _():
        counts = counts_ref[...]
        sums = jnp.sum(sums_ref[...], axis=0) + tcsums_ref[...]
        sums_v[...] = sums
        nc_ref[0] = jnp.sum((counts > 0.0).astype(jnp.int32))
        cent_ref[...] = sums / jnp.maximum(counts, 1.0)
        acc_ref[0] = 0.0

    nc = nc_ref[0]
    g = jnp.minimum(lab, nc - 1)
    onehot = (g[:, None] ==
              lax.broadcasted_iota(jnp.int32, (_R, _C), 1)
              ).astype(jnp.float32)
    cent_rows = jnp.dot(onehot, cent_ref[...],
                        preferred_element_type=jnp.float32)
    diff = x_ref[...] - cent_rows
    ones_mat = jnp.ones((_D, _D), dtype=jnp.float32)
    e_dup = jnp.dot(diff * diff, ones_mat,
                    preferred_element_type=jnp.float32)
    dist = e_dup * lax.rsqrt(e_dup + 1e-37)
    acc_ref[0] += jnp.sum(dist) * (1.0 / float(_D))

    @pl.when(i == nb - 1)
    def _():
        nc_f = nc.astype(jnp.float32)
        counts = counts_ref[...]
        gm = jnp.sum(sums_v[...], axis=0, keepdims=True) / float(_N)
        dc = cent_ref[...] - gm
        e_c = jnp.sum(dc * dc, axis=1, keepdims=True)
        d = e_c * lax.rsqrt(e_c + 1e-37)
        cidx = lax.broadcasted_iota(jnp.int32, (_C, 1), 0)
        between = (jnp.sum(jnp.where(cidx < nc, counts * d, 0.0))
                   / (nc_f - 1.0))
        within = acc_ref[0] / (float(_N) - nc_f)
        out_ref[...] = jnp.full((1, 1), between / within, dtype=jnp.float32)


def _distance_pass(sums2, tcsums, counts, x, labels):
    nb = _N // _R
    lab3 = labels.reshape(nb, 1, _R)
    sums3 = sums2.reshape(_NW, _C, _D)
    return pl.pallas_call(
        _dist_kernel,
        grid=(nb,),
        in_specs=[
            pl.BlockSpec((_NW, _C, _D), lambda i: (0, 0, 0)),
            pl.BlockSpec((_C, _D), lambda i: (0, 0)),
            pl.BlockSpec((_C, 1), lambda i: (0, 0)),
            pl.BlockSpec((_R, _D), lambda i: (i, 0)),
            pl.BlockSpec((1, 1, _R), lambda i: (i, 0, 0)),
        ],
        out_specs=pl.BlockSpec((1, 1), lambda i: (0, 0)),
        out_shape=jax.ShapeDtypeStruct((1, 1), jnp.float32),
        scratch_shapes=[
            pltpu.VMEM((_C, _D), jnp.float32),
            pltpu.VMEM((_C, _D), jnp.float32),
            pltpu.SMEM((1,), jnp.float32),
            pltpu.SMEM((1,), jnp.int32),
        ],
    )(sums3, tcsums, counts, x, lab3)


def kernel(Attributes, cluster_labels):
    labels = cluster_labels[0]
    sums2 = _sc_segment_sums(Attributes, labels)
    tcsums, counts = _tc_sega(Attributes, labels)
    loss = _distance_pass(sums2, tcsums, counts, Attributes, labels)
    return loss.reshape(1)
